# baseline (speedup 1.0000x reference)
import jax
import jax.numpy as jnp
from jax import lax
from jax.experimental import pallas as pl


def _vq_body(xa_ref, wt2_ref, w2_ref, w_ref, idx_ref, q_ref):
    xa = xa_ref[0]
    wt2 = wt2_ref[...]
    d = wt2.shape[0] - 1
    p2 = lax.dot_general(xa, wt2, (((1,), (0,)), ((), ())),
                         preferred_element_type=jnp.float32)
    x2 = xa[:, d:]
    d2 = (x2 + w2_ref[...]) - p2
    dist = jnp.sqrt(jnp.maximum(d2, 0.0))
    k = dist.shape[1]
    m = jnp.min(dist, axis=1, keepdims=True)
    ksf = lax.broadcasted_iota(jnp.int32, dist.shape, 1).astype(jnp.float32)
    idxf = jnp.min(jnp.where(dist == m, ksf, float(k)), axis=1)
    idx_ref[0, 0] = idxf.astype(jnp.int32)
    onehot = (ksf == idxf[:, None]).astype(jnp.float32)
    q_ref[0] = lax.dot_general(onehot, w_ref[...], (((1,), (0,)), ((), ())),
                               preferred_element_type=jnp.float32)


def kernel(x, W):
    b, n, d = x.shape
    nk = W.shape[0]
    x2 = jnp.sum(x * x, axis=-1, keepdims=True)
    w2_row = jnp.sum(W * W, axis=-1)[None, :]
    xa = jnp.concatenate([x, x2], axis=-1)
    wt2 = jnp.concatenate([(2.0 * W).T, jnp.zeros((1, nk), jnp.float32)],
                          axis=0)
    idx, quant = pl.pallas_call(
        _vq_body,
        grid=(b,),
        in_specs=[
            pl.BlockSpec((1, n, d + 1), lambda i: (i, 0, 0)),
            pl.BlockSpec((d + 1, nk), lambda i: (0, 0)),
            pl.BlockSpec((1, nk), lambda i: (0, 0)),
            pl.BlockSpec((nk, d), lambda i: (0, 0)),
        ],
        out_specs=[
            pl.BlockSpec((1, 1, n), lambda i: (i, 0, 0)),
            pl.BlockSpec((1, n, d), lambda i: (i, 0, 0)),
        ],
        out_shape=[
            jax.ShapeDtypeStruct((b, 1, n), jnp.int32),
            jax.ShapeDtypeStruct((b, n, d), jnp.float32),
        ],
    )(xa, wt2, w2_row, W)
    return quant, idx.reshape(b, n)


# speedup vs baseline: 1.1532x; 1.1532x over previous
import jax
import jax.numpy as jnp
from jax import lax
from jax.experimental import pallas as pl


def _vq_body(x_ref, x2_ref, wt2_ref, w2_ref, w_ref, idx_ref, q_ref):
    x = x_ref[0]
    wt2 = wt2_ref[...]
    p2 = lax.dot_general(x, wt2, (((1,), (0,)), ((), ())),
                         preferred_element_type=jnp.float32)
    x2 = lax.transpose(x2_ref[0], (1, 0))
    d2 = (x2 + w2_ref[...]) - p2
    dist = jnp.sqrt(jnp.maximum(d2, 0.0))
    k = dist.shape[1]
    m = jnp.min(dist, axis=1, keepdims=True)
    ksf = lax.broadcasted_iota(jnp.int32, dist.shape, 1).astype(jnp.float32)
    idxf = jnp.min(jnp.where(dist == m, ksf, float(k)), axis=1)
    idx_ref[0, 0] = idxf.astype(jnp.int32)
    onehot = (ksf == idxf[:, None]).astype(jnp.float32)
    q_ref[0] = lax.dot_general(onehot, w_ref[...], (((1,), (0,)), ((), ())),
                               preferred_element_type=jnp.float32)


def kernel(x, W):
    b, n, d = x.shape
    nk = W.shape[0]
    x2_row = jnp.sum(x * x, axis=-1)[:, None, :]
    w2_row = jnp.sum(W * W, axis=-1)[None, :]
    wt2 = (2.0 * W).T
    idx, quant = pl.pallas_call(
        _vq_body,
        grid=(b,),
        in_specs=[
            pl.BlockSpec((1, n, d), lambda i: (i, 0, 0)),
            pl.BlockSpec((1, 1, n), lambda i: (i, 0, 0)),
            pl.BlockSpec((d, nk), lambda i: (0, 0)),
            pl.BlockSpec((1, nk), lambda i: (0, 0)),
            pl.BlockSpec((nk, d), lambda i: (0, 0)),
        ],
        out_specs=[
            pl.BlockSpec((1, 1, n), lambda i: (i, 0, 0)),
            pl.BlockSpec((1, n, d), lambda i: (i, 0, 0)),
        ],
        out_shape=[
            jax.ShapeDtypeStruct((b, 1, n), jnp.int32),
            jax.ShapeDtypeStruct((b, n, d), jnp.float32),
        ],
    )(x, x2_row, wt2, w2_row, W)
    return quant, idx.reshape(b, n)


# speedup vs baseline: 1.2938x; 1.1219x over previous
import jax
import jax.numpy as jnp
from jax import lax
from jax.experimental import pallas as pl


def _vq_body(xt_ref, x2_ref, wt2_ref, w2_ref, wt_ref, idx_ref, qt_ref):
    xt = xt_ref[0]
    wt2 = wt2_ref[...]
    p2t = lax.dot_general(wt2, xt, (((0,), (0,)), ((), ())),
                          preferred_element_type=jnp.float32)
    x2 = x2_ref[0]
    w2 = lax.transpose(w2_ref[...], (1, 0))
    d2 = (x2 + w2) - p2t
    dist = jnp.sqrt(jnp.maximum(d2, 0.0))
    k = dist.shape[0]
    m = jnp.min(dist, axis=0, keepdims=True)
    ksf = lax.broadcasted_iota(jnp.int32, dist.shape, 0).astype(jnp.float32)
    idxf = jnp.min(jnp.where(dist == m, ksf, float(k)), axis=0)
    idx_ref[0, 0] = idxf.astype(jnp.int32)
    onehot_t = (ksf == idxf[None, :]).astype(jnp.float32)
    qt_ref[0] = lax.dot_general(wt_ref[...], onehot_t, (((1,), (0,)), ((), ())),
                                preferred_element_type=jnp.float32)


def kernel(x, W):
    b, n, d = x.shape
    nk = W.shape[0]
    xt = x.transpose(0, 2, 1)
    x2_row = jnp.sum(x * x, axis=-1)[:, None, :]
    w2_row = jnp.sum(W * W, axis=-1)[None, :]
    wt2 = (2.0 * W).T
    wt = W.T
    idx, qt = pl.pallas_call(
        _vq_body,
        grid=(b,),
        in_specs=[
            pl.BlockSpec((1, d, n), lambda i: (i, 0, 0)),
            pl.BlockSpec((1, 1, n), lambda i: (i, 0, 0)),
            pl.BlockSpec((d, nk), lambda i: (0, 0)),
            pl.BlockSpec((1, nk), lambda i: (0, 0)),
            pl.BlockSpec((d, nk), lambda i: (0, 0)),
        ],
        out_specs=[
            pl.BlockSpec((1, 1, n), lambda i: (i, 0, 0)),
            pl.BlockSpec((1, d, n), lambda i: (i, 0, 0)),
        ],
        out_shape=[
            jax.ShapeDtypeStruct((b, 1, n), jnp.int32),
            jax.ShapeDtypeStruct((b, d, n), jnp.float32),
        ],
    )(xt, x2_row, wt2, w2_row, wt)
    return qt.transpose(0, 2, 1), idx.reshape(b, n)


# speedup vs baseline: 1.4260x; 1.1022x over previous
import jax
import jax.numpy as jnp
from jax import lax
from jax.experimental import pallas as pl


def _vq_body(xt_ref, wt_ref, idx_ref, qt_ref):
    xt = xt_ref[0]
    wt = wt_ref[...]
    x2 = jnp.sum(xt * xt, axis=0, keepdims=True)
    w2 = lax.transpose(jnp.sum(wt * wt, axis=0, keepdims=True), (1, 0))
    p2t = lax.dot_general(wt + wt, xt, (((0,), (0,)), ((), ())),
                          preferred_element_type=jnp.float32)
    d2 = (x2 + w2) - p2t
    dist = jnp.sqrt(jnp.maximum(d2, 0.0))
    k = dist.shape[0]
    m = jnp.min(dist, axis=0, keepdims=True)
    ksf = lax.broadcasted_iota(jnp.int32, dist.shape, 0).astype(jnp.float32)
    idxf = jnp.min(jnp.where(dist == m, ksf, float(k)), axis=0)
    idx_ref[0, 0] = idxf.astype(jnp.int32)
    onehot_t = (ksf == idxf[None, :]).astype(jnp.float32)
    qt_ref[0] = lax.dot_general(wt, onehot_t, (((1,), (0,)), ((), ())),
                                preferred_element_type=jnp.float32)


def kernel(x, W):
    b, n, d = x.shape
    nk = W.shape[0]
    xt = x.transpose(0, 2, 1)
    wt = W.T
    idx, qt = pl.pallas_call(
        _vq_body,
        grid=(b,),
        in_specs=[
            pl.BlockSpec((1, d, n), lambda i: (i, 0, 0)),
            pl.BlockSpec((d, nk), lambda i: (0, 0)),
        ],
        out_specs=[
            pl.BlockSpec((1, 1, n), lambda i: (i, 0, 0)),
            pl.BlockSpec((1, d, n), lambda i: (i, 0, 0)),
        ],
        out_shape=[
            jax.ShapeDtypeStruct((b, 1, n), jnp.int32),
            jax.ShapeDtypeStruct((b, d, n), jnp.float32),
        ],
    )(xt, wt)
    return qt.transpose(0, 2, 1), idx.reshape(b, n)


# speedup vs baseline: 1.5781x; 1.1067x over previous
import jax
import jax.numpy as jnp
from jax import lax
from jax.experimental import pallas as pl


BATCHES_PER_STEP = 2


def _vq_body(xt_ref, wt_ref, idx_ref, qt_ref):
    wt = wt_ref[...]
    w2 = lax.transpose(jnp.sum(wt * wt, axis=0, keepdims=True), (1, 0))
    wt2 = wt + wt
    for j in range(BATCHES_PER_STEP):
        xt = xt_ref[j]
        x2 = jnp.sum(xt * xt, axis=0, keepdims=True)
        p2t = lax.dot_general(wt2, xt, (((0,), (0,)), ((), ())),
                              preferred_element_type=jnp.float32)
        d2 = (x2 + w2) - p2t
        dist = jnp.sqrt(jnp.maximum(d2, 0.0))
        k = dist.shape[0]
        m = jnp.min(dist, axis=0, keepdims=True)
        ksf = lax.broadcasted_iota(jnp.int32, dist.shape, 0).astype(jnp.float32)
        idxf = jnp.min(jnp.where(dist == m, ksf, float(k)), axis=0)
        idx_ref[j, 0] = idxf.astype(jnp.int32)
        onehot_t = (ksf == idxf[None, :]).astype(jnp.float32)
        qt_ref[j] = lax.dot_general(wt, onehot_t, (((1,), (0,)), ((), ())),
                                    preferred_element_type=jnp.float32)


def kernel(x, W):
    b, n, d = x.shape
    nk = W.shape[0]
    xt = x.transpose(0, 2, 1)
    wt = W.T
    g = BATCHES_PER_STEP
    idx, qt = pl.pallas_call(
        _vq_body,
        grid=(b // g,),
        in_specs=[
            pl.BlockSpec((g, d, n), lambda i: (i, 0, 0)),
            pl.BlockSpec((d, nk), lambda i: (0, 0)),
        ],
        out_specs=[
            pl.BlockSpec((g, 1, n), lambda i: (i, 0, 0)),
            pl.BlockSpec((g, d, n), lambda i: (i, 0, 0)),
        ],
        out_shape=[
            jax.ShapeDtypeStruct((b, 1, n), jnp.int32),
            jax.ShapeDtypeStruct((b, d, n), jnp.float32),
        ],
    )(xt, wt)
    return qt.transpose(0, 2, 1), idx.reshape(b, n)


# speedup vs baseline: 1.6880x; 1.0696x over previous
import jax
import jax.numpy as jnp
from jax import lax
from jax.experimental import pallas as pl


BATCHES_PER_STEP = 4


def _vq_body(xt_ref, wt_ref, idx_ref, qt_ref):
    wt = wt_ref[...]
    w2 = lax.transpose(jnp.sum(wt * wt, axis=0, keepdims=True), (1, 0))
    wt2 = wt + wt
    for j in range(BATCHES_PER_STEP):
        xt = xt_ref[j]
        x2 = jnp.sum(xt * xt, axis=0, keepdims=True)
        p2t = lax.dot_general(wt2, xt, (((0,), (0,)), ((), ())),
                              preferred_element_type=jnp.float32)
        d2 = (x2 + w2) - p2t
        dist = jnp.sqrt(jnp.maximum(d2, 0.0))
        k = dist.shape[0]
        m = jnp.min(dist, axis=0, keepdims=True)
        ksf = lax.broadcasted_iota(jnp.int32, dist.shape, 0).astype(jnp.float32)
        idxf = jnp.min(jnp.where(dist == m, ksf, float(k)), axis=0)
        idx_ref[j, 0] = idxf.astype(jnp.int32)
        onehot_t = (ksf == idxf[None, :]).astype(jnp.float32)
        qt_ref[j] = lax.dot_general(wt, onehot_t, (((1,), (0,)), ((), ())),
                                    preferred_element_type=jnp.float32)


def kernel(x, W):
    b, n, d = x.shape
    nk = W.shape[0]
    xt = x.transpose(0, 2, 1)
    wt = W.T
    g = BATCHES_PER_STEP
    idx, qt = pl.pallas_call(
        _vq_body,
        grid=(b // g,),
        in_specs=[
            pl.BlockSpec((g, d, n), lambda i: (i, 0, 0)),
            pl.BlockSpec((d, nk), lambda i: (0, 0)),
        ],
        out_specs=[
            pl.BlockSpec((g, 1, n), lambda i: (i, 0, 0)),
            pl.BlockSpec((g, d, n), lambda i: (i, 0, 0)),
        ],
        out_shape=[
            jax.ShapeDtypeStruct((b, 1, n), jnp.int32),
            jax.ShapeDtypeStruct((b, d, n), jnp.float32),
        ],
    )(xt, wt)
    return qt.transpose(0, 2, 1), idx.reshape(b, n)


# speedup vs baseline: 1.7202x; 1.0191x over previous
import jax
import jax.numpy as jnp
from jax import lax
from jax.experimental import pallas as pl


BATCHES_PER_STEP = 8


def _vq_body(xt_ref, wt_ref, idx_ref, qt_ref):
    wt = wt_ref[...]
    w2 = lax.transpose(jnp.sum(wt * wt, axis=0, keepdims=True), (1, 0))
    wt2 = wt + wt
    for j in range(BATCHES_PER_STEP):
        xt = xt_ref[j]
        x2 = jnp.sum(xt * xt, axis=0, keepdims=True)
        p2t = lax.dot_general(wt2, xt, (((0,), (0,)), ((), ())),
                              preferred_element_type=jnp.float32)
        d2 = (x2 + w2) - p2t
        dist = jnp.sqrt(jnp.maximum(d2, 0.0))
        k = dist.shape[0]
        m = jnp.min(dist, axis=0, keepdims=True)
        ksf = lax.broadcasted_iota(jnp.int32, dist.shape, 0).astype(jnp.float32)
        idxf = jnp.min(jnp.where(dist == m, ksf, float(k)), axis=0)
        idx_ref[j, 0] = idxf.astype(jnp.int32)
        onehot_t = (ksf == idxf[None, :]).astype(jnp.float32)
        qt_ref[j] = lax.dot_general(wt, onehot_t, (((1,), (0,)), ((), ())),
                                    preferred_element_type=jnp.float32)


def kernel(x, W):
    b, n, d = x.shape
    nk = W.shape[0]
    xt = x.transpose(0, 2, 1)
    wt = W.T
    g = BATCHES_PER_STEP
    idx, qt = pl.pallas_call(
        _vq_body,
        grid=(b // g,),
        in_specs=[
            pl.BlockSpec((g, d, n), lambda i: (i, 0, 0)),
            pl.BlockSpec((d, nk), lambda i: (0, 0)),
        ],
        out_specs=[
            pl.BlockSpec((g, 1, n), lambda i: (i, 0, 0)),
            pl.BlockSpec((g, d, n), lambda i: (i, 0, 0)),
        ],
        out_shape=[
            jax.ShapeDtypeStruct((b, 1, n), jnp.int32),
            jax.ShapeDtypeStruct((b, d, n), jnp.float32),
        ],
    )(xt, wt)
    return qt.transpose(0, 2, 1), idx.reshape(b, n)
